# Initial kernel scaffold; baseline (speedup 1.0000x reference)
import functools

import jax
import jax.numpy as jnp
from jax import lax
from jax.experimental import pallas as pl
from jax.experimental.pallas import tpu as pltpu
from jax.experimental.pallas import tpu_sc as plsc

VOCAB = 1000000
BATCH = 16384
M = 100

NUM_WORKERS = 32
BPW = BATCH // NUM_WORKERS
IPW = BPW * M
IDX_MINOR = 128
IDX_CHUNKS = IPW // IDX_MINOR
GROUPS = BPW // 16

_mesh = plsc.VectorSubcoreMesh(core_axis_name="c", subcore_axis_name="s")


@functools.partial(
    pl.kernel,
    mesh=_mesh,
    out_type=jax.ShapeDtypeStruct((BATCH,), jnp.float32),
    scratch_types=[
        pltpu.VMEM((IDX_CHUNKS, IDX_MINOR), jnp.int32),
        pltpu.VMEM((IDX_CHUNKS, IDX_MINOR), jnp.float32),
        pltpu.VMEM((BPW,), jnp.float32),
        pltpu.SemaphoreType.DMA,
    ],
)
def _emb_sum(idx_hbm, tbl_hbm, out_hbm, idx_v, vals_v, out_v, sem):
    wid = lax.axis_index("s") * 2 + lax.axis_index("c")
    pltpu.sync_copy(idx_hbm.at[wid], idx_v)
    pltpu.async_copy(tbl_hbm.at[idx_v], vals_v, sem).wait()

    lane_off = lax.iota(jnp.int32, (16,)) * M

    for g in range(GROUPS):
        e0 = lane_off + (g * 16 * M)

        def body(m, carry):
            acc, e = carry
            v = plsc.load_gather(
                vals_v, [lax.shift_right_logical(e, 7), lax.bitwise_and(e, 127)]
            )
            return acc + v, e + 1

        acc, _ = lax.fori_loop(
            0, M, body, (jnp.zeros((16,), jnp.float32), e0)
        )
        out_v[pl.ds(g * 16, 16)] = acc

    pltpu.sync_copy(out_v, out_hbm.at[pl.ds(wid * BPW, BPW)])


def kernel(inputs, linear_weights):
    idx = inputs.astype(jnp.int32).reshape(NUM_WORKERS, IDX_CHUNKS, IDX_MINOR)
    tbl = jnp.concatenate(
        [jnp.zeros((1,), jnp.float32), linear_weights.reshape(VOCAB)]
    )
    out = _emb_sum(idx, tbl)
    return out.reshape(BATCH, 1)


# speedup vs baseline: 1.3082x; 1.3082x over previous
import functools

import jax
import jax.numpy as jnp
from jax import lax
from jax.experimental import pallas as pl
from jax.experimental.pallas import tpu as pltpu
from jax.experimental.pallas import tpu_sc as plsc

VOCAB = 1000000
BATCH = 16384
M = 100

NUM_WORKERS = 32
BPW = BATCH // NUM_WORKERS
IPW = BPW * M
GROUPS = BPW // 16

_mesh = plsc.VectorSubcoreMesh(core_axis_name="c", subcore_axis_name="s")


@functools.partial(
    pl.kernel,
    mesh=_mesh,
    out_type=jax.ShapeDtypeStruct((BATCH,), jnp.float32),
    scratch_types=[
        pltpu.VMEM((IPW,), jnp.int32),
        pltpu.VMEM((IPW,), jnp.float32),
        pltpu.VMEM((BPW,), jnp.float32),
        pltpu.SemaphoreType.DMA,
    ],
)
def _emb_sum(idx_hbm, tbl_hbm, out_hbm, idx_v, vals_v, out_v, sem):
    wid = lax.axis_index("s") * 2 + lax.axis_index("c")
    pltpu.sync_copy(idx_hbm.at[wid], idx_v)
    pltpu.async_copy(tbl_hbm.at[idx_v], vals_v, sem).wait()

    def body(m, accs):
        base = m * BPW
        return tuple(
            accs[g] + vals_v[pl.ds(base + g * 16, 16)] for g in range(GROUPS)
        )

    zero = jnp.zeros((16,), jnp.float32)
    accs = lax.fori_loop(0, M, body, (zero,) * GROUPS)
    for g in range(GROUPS):
        out_v[pl.ds(g * 16, 16)] = accs[g]

    pltpu.sync_copy(out_v, out_hbm.at[pl.ds(wid * BPW, BPW)])


def kernel(inputs, linear_weights):
    idx = (
        (inputs.astype(jnp.int32) - 1)
        .reshape(NUM_WORKERS, BPW, M)
        .transpose(0, 2, 1)
        .reshape(NUM_WORKERS, IPW)
    )
    out = _emb_sum(idx, linear_weights.reshape(VOCAB))
    return out.reshape(BATCH, 1)


# speedup vs baseline: 1.7775x; 1.3588x over previous
import functools

import jax
import jax.numpy as jnp
from jax import lax
from jax.experimental import pallas as pl
from jax.experimental.pallas import tpu as pltpu
from jax.experimental.pallas import tpu_sc as plsc

VOCAB = 1000000
BATCH = 16384
M = 100
VP = 1001472

NUM_WORKERS = 32
BPW = BATCH // NUM_WORKERS
IPW = BPW * M

_mesh = plsc.VectorSubcoreMesh(core_axis_name="c", subcore_axis_name="s")


@functools.partial(
    pl.kernel,
    mesh=_mesh,
    compiler_params=pltpu.CompilerParams(needs_layout_passes=False),
    out_type=jax.ShapeDtypeStruct((BATCH,), jnp.float32),
    scratch_types=[
        pltpu.VMEM((IPW,), jnp.int32),
        pltpu.VMEM((IPW + 16,), jnp.float32),
        pltpu.VMEM((BPW,), jnp.float32),
        pltpu.SemaphoreType.DMA,
    ],
)
def _emb_sum(idx_hbm, tbl_hbm, out_hbm, idx_v, vals_v, out_v, sem):
    wid = lax.axis_index("s") * 2 + lax.axis_index("c")
    pltpu.sync_copy(idx_hbm.at[wid], idx_v)
    pltpu.async_copy(tbl_hbm.at[idx_v], vals_v.at[pl.ds(0, IPW)], sem).wait()

    lane = lax.iota(jnp.int32, 16)
    tail_mask = lane < (M - 96)

    def grp_body(g, carry):
        base0 = g * (16 * M)
        res = jnp.zeros((16,), jnp.float32)
        for i in range(16):
            base = base0 + i * M
            s = vals_v[pl.ds(base, 16)]
            for k in range(1, 6):
                s = s + vals_v[pl.ds(base + 16 * k, 16)]
            t = vals_v[pl.ds(base + 96, 16)]
            s = s + jnp.where(tail_mask, t, 0.0)
            res = jnp.where(lane == i, jnp.sum(s), res)
        out_v[pl.ds(g * 16, 16)] = res
        return carry

    lax.fori_loop(0, BPW // 16, grp_body, 0)

    pltpu.sync_copy(out_v, out_hbm.at[pl.ds(wid * BPW, BPW)])


def kernel(inputs, linear_weights):
    idx = (inputs.astype(jnp.int32) - 1).reshape(NUM_WORKERS, IPW)
    tbl = jnp.pad(linear_weights, ((0, VP - VOCAB), (0, 0))).reshape(VP)
    out = _emb_sum(idx, tbl)
    return out.reshape(BATCH, 1)
